# baseline (speedup 1.0000x reference)
import functools

import jax
import jax.numpy as jnp
from jax import lax
from jax.experimental import pallas as pl
from jax.experimental.pallas import tpu as pltpu
from jax.experimental.pallas import tpu_sc as plsc

_BATCH = 4
_TOTAL = 4160
_DIM = 1024
_NC = 2
_NS = 16
_NW = _NC * _NS
_ROWS_PER_W = _TOTAL // _NW
_C = 26
_CHUNKS = _ROWS_PER_W // _C
_CW = _C * _DIM
_LANES = 16
_UNROLL = 8


def _sc_body(in_hbm, tab_hbm, out_hbm, tab_v, io_v, sem):
    wid = lax.axis_index("s") * _NC + lax.axis_index("c")
    base = wid * _ROWS_PER_W * _DIM

    def add_chunk(_, __):
        def vec_body(i, _):
            off = i * (_LANES * _UNROLL)
            for u in range(_UNROLL):
                ds = pl.ds(off + u * _LANES, _LANES)
                io_v[ds] = io_v[ds] + tab_v[ds]
            return 0

        return lax.fori_loop(0, _CW // (_LANES * _UNROLL), vec_body, 0)

    for c in range(_CHUNKS):
        off = base + c * _CW
        pltpu.sync_copy(tab_hbm.at[pl.ds(off, _CW)], tab_v)
        for b in range(_BATCH):
            boff = b * _TOTAL * _DIM + off
            pltpu.sync_copy(in_hbm.at[pl.ds(boff, _CW)], io_v)
            add_chunk(0, 0)
            pltpu.sync_copy(io_v, out_hbm.at[pl.ds(boff, _CW)])


def kernel(inputs, table):
    mesh = plsc.VectorSubcoreMesh(
        core_axis_name="c", subcore_axis_name="s"
    )
    run = pl.kernel(
        _sc_body,
        out_type=jax.ShapeDtypeStruct((_BATCH * _TOTAL * _DIM,), jnp.float32),
        mesh=mesh,
        scratch_types=[
            pltpu.VMEM((_CW,), jnp.float32),
            pltpu.VMEM((_CW,), jnp.float32),
            pltpu.SemaphoreType.DMA,
        ],
    )
    out = run(inputs.reshape(-1), table.reshape(-1))
    return out.reshape(inputs.shape)


# speedup vs baseline: 1.1950x; 1.1950x over previous
import jax
import jax.numpy as jnp
from jax import lax
from jax.experimental import pallas as pl
from jax.experimental.pallas import tpu as pltpu
from jax.experimental.pallas import tpu_sc as plsc

_BATCH = 4
_TOTAL = 4160
_DIM = 1024
_NC = 2
_NS = 16
_NW = _NC * _NS
_ROWS_PER_W = _TOTAL // _NW
_C = 13
_CHUNKS = _ROWS_PER_W // _C
_CW = _C * _DIM
_STEPS = _CHUNKS * _BATCH
_LANES = 16
_UNROLL = 8


def _sc_body(in_hbm, tab_hbm, out_hbm,
             tab0, tab1, in0, in1, out0, out1,
             tab_sems, in_sems, out_sems):
    wid = lax.axis_index("s") * _NC + lax.axis_index("c")
    base = wid * (_ROWS_PER_W * _DIM)
    tabs, ins, outs = (tab0, tab1), (in0, in1), (out0, out1)

    def tab_src(c):
        return tab_hbm.at[pl.ds(base + c * _CW, _CW)]

    def io_slice(s):
        c, b = divmod(s, _BATCH)
        off = base + b * (_TOTAL * _DIM) + c * _CW
        return pl.ds(off, _CW)

    def add_chunk(src, tab, dst):
        def vec_body(i, _):
            off = i * (_LANES * _UNROLL)
            for u in range(_UNROLL):
                ds = pl.ds(off + u * _LANES, _LANES)
                dst[ds] = src[ds] + tab[ds]
            return 0

        lax.fori_loop(0, _CW // (_LANES * _UNROLL), vec_body, 0)

    tab_dma = {}
    in_dma = {}
    out_dma = {}

    tab_dma[0] = pltpu.make_async_copy(tab_src(0), tabs[0], tab_sems.at[0])
    tab_dma[0].start()
    for s in (0, 1):
        in_dma[s] = pltpu.make_async_copy(
            in_hbm.at[io_slice(s)], ins[s % 2], in_sems.at[s % 2])
        in_dma[s].start()

    for s in range(_STEPS):
        c, b = divmod(s, _BATCH)
        if b == _BATCH - 1 and c + 1 < _CHUNKS:
            tab_dma[c + 1] = pltpu.make_async_copy(
                tab_src(c + 1), tabs[(c + 1) % 2], tab_sems.at[(c + 1) % 2])
            tab_dma[c + 1].start()
        in_dma[s].wait()
        if b == 0:
            tab_dma[c].wait()
        if s >= 2:
            out_dma[s - 2].wait()
        add_chunk(ins[s % 2], tabs[c % 2], outs[s % 2])
        out_dma[s] = pltpu.make_async_copy(
            outs[s % 2], out_hbm.at[io_slice(s)], out_sems.at[s % 2])
        out_dma[s].start()
        if s + 2 < _STEPS:
            in_dma[s + 2] = pltpu.make_async_copy(
                in_hbm.at[io_slice(s + 2)], ins[s % 2], in_sems.at[s % 2])
            in_dma[s + 2].start()

    out_dma[_STEPS - 2].wait()
    out_dma[_STEPS - 1].wait()


def kernel(inputs, table):
    mesh = plsc.VectorSubcoreMesh(
        core_axis_name="c", subcore_axis_name="s"
    )
    run = pl.kernel(
        _sc_body,
        out_type=jax.ShapeDtypeStruct((_BATCH * _TOTAL * _DIM,), jnp.float32),
        mesh=mesh,
        scratch_types=[
            pltpu.VMEM((_CW,), jnp.float32),
            pltpu.VMEM((_CW,), jnp.float32),
            pltpu.VMEM((_CW,), jnp.float32),
            pltpu.VMEM((_CW,), jnp.float32),
            pltpu.VMEM((_CW,), jnp.float32),
            pltpu.VMEM((_CW,), jnp.float32),
            pltpu.SemaphoreType.DMA((2,)),
            pltpu.SemaphoreType.DMA((2,)),
            pltpu.SemaphoreType.DMA((2,)),
        ],
    )
    out = run(inputs.reshape(-1), table.reshape(-1))
    return out.reshape(inputs.shape)


# speedup vs baseline: 2.8564x; 2.3902x over previous
import jax
import jax.numpy as jnp
from jax import lax
from jax.experimental import pallas as pl
from jax.experimental.pallas import tpu as pltpu
from jax.experimental.pallas import tpu_sc as plsc

_BATCH = 4
_TOTAL = 4160
_DIM = 1024
_NC = 2
_NS = 16
_NW = _NC * _NS
_GROUPS = _TOTAL // 8
_BASE_GROUPS = _GROUPS // _NW
_EXTRA_WORKERS = _GROUPS % _NW
_C = 16
_CHUNKS = (_BASE_GROUPS * 8) // _C
_STEPS = _CHUNKS * _BATCH
_LANES = 16
_UNROLL = 8


def _add_rows(src, tab, dst, rows):
    def vec_body(i, _):
        r = i >> 3
        j0 = (i & 7) * (_LANES * _UNROLL)
        for u in range(_UNROLL):
            ds = pl.ds(j0 + u * _LANES, _LANES)
            dst[r, ds] = src[r, ds] + tab[r, ds]
        return 0

    lax.fori_loop(0, rows * (_DIM // (_LANES * _UNROLL)), vec_body, 0)


def _sc_body(in_hbm, tab_hbm, out_hbm,
             tab0, tab1, in0, in1, out0, out1,
             tab_sems, in_sems, out_sems):
    wid = lax.axis_index("s") * _NC + lax.axis_index("c")
    base_row = 8 * (_BASE_GROUPS * wid + jnp.minimum(wid, _EXTRA_WORKERS))
    tabs, ins, outs = (tab0, tab1), (in0, in1), (out0, out1)

    def tab_slice(c):
        return (pl.ds(base_row + c * _C, _C), slice(None))

    def io_slice(s):
        c, b = divmod(s, _BATCH)
        return (pl.ds(b * _TOTAL + base_row + c * _C, _C), slice(None))

    tab_dma = {}
    in_dma = {}
    out_dma = {}

    tab_dma[0] = pltpu.make_async_copy(
        tab_hbm.at[tab_slice(0)], tabs[0], tab_sems.at[0])
    tab_dma[0].start()
    for s in (0, 1):
        in_dma[s] = pltpu.make_async_copy(
            in_hbm.at[io_slice(s)], ins[s % 2], in_sems.at[s % 2])
        in_dma[s].start()

    for s in range(_STEPS):
        c, b = divmod(s, _BATCH)
        if b == _BATCH - 1 and c + 1 < _CHUNKS:
            tab_dma[c + 1] = pltpu.make_async_copy(
                tab_hbm.at[tab_slice(c + 1)], tabs[(c + 1) % 2],
                tab_sems.at[(c + 1) % 2])
            tab_dma[c + 1].start()
        in_dma[s].wait()
        if b == 0:
            tab_dma[c].wait()
        if s >= 2:
            out_dma[s - 2].wait()
        _add_rows(ins[s % 2], tabs[c % 2], outs[s % 2], _C)
        out_dma[s] = pltpu.make_async_copy(
            outs[s % 2], out_hbm.at[io_slice(s)], out_sems.at[s % 2])
        out_dma[s].start()
        if s + 2 < _STEPS:
            in_dma[s + 2] = pltpu.make_async_copy(
                in_hbm.at[io_slice(s + 2)], ins[s % 2], in_sems.at[s % 2])
            in_dma[s + 2].start()

    out_dma[_STEPS - 2].wait()
    out_dma[_STEPS - 1].wait()

    @pl.when(wid < _EXTRA_WORKERS)
    def _tail():
        row0 = base_row + _BASE_GROUPS * 8
        pltpu.sync_copy(tab_hbm.at[pl.ds(row0, 8), :],
                        tabs[0].at[pl.ds(0, 8), :])
        for b in range(_BATCH):
            sl = (pl.ds(b * _TOTAL + row0, 8), slice(None))
            pltpu.sync_copy(in_hbm.at[sl], ins[0].at[pl.ds(0, 8), :])
            _add_rows(ins[0], tabs[0], outs[0], 8)
            pltpu.sync_copy(outs[0].at[pl.ds(0, 8), :], out_hbm.at[sl])


def kernel(inputs, table):
    mesh = plsc.VectorSubcoreMesh(
        core_axis_name="c", subcore_axis_name="s"
    )
    run = pl.kernel(
        _sc_body,
        out_type=jax.ShapeDtypeStruct((_BATCH * _TOTAL, _DIM), jnp.float32),
        mesh=mesh,
        scratch_types=[
            pltpu.VMEM((_C, _DIM), jnp.float32),
            pltpu.VMEM((_C, _DIM), jnp.float32),
            pltpu.VMEM((_C, _DIM), jnp.float32),
            pltpu.VMEM((_C, _DIM), jnp.float32),
            pltpu.VMEM((_C, _DIM), jnp.float32),
            pltpu.VMEM((_C, _DIM), jnp.float32),
            pltpu.SemaphoreType.DMA((2,)),
            pltpu.SemaphoreType.DMA((2,)),
            pltpu.SemaphoreType.DMA((2,)),
        ],
        compiler_params=pltpu.CompilerParams(use_tc_tiling_on_sc=True),
    )
    out = run(inputs.reshape(_BATCH * _TOTAL, _DIM), table)
    return out.reshape(inputs.shape)
